# baseline (speedup 1.0000x reference)
import jax
import jax.numpy as jnp
from jax.experimental import pallas as pl
from jax.experimental.pallas import tpu as pltpu


def _round_up(x, m):
    return ((x + m - 1) // m) * m


def _cdiv(a, b):
    return (a + b - 1) // b


def _make_kernel(fh):
    def _mod_linear_kernel(xa_ref, xb_ref, theta_ref, ga_ref, gb_ref,
                           bias_ref, out_ref):
        xs_a = (xa_ref[...] * theta_ref[:, :fh]).astype(jnp.bfloat16)
        xs_b = (xb_ref[...] * theta_ref[:, fh:]).astype(jnp.bfloat16)
        dn = (((1,), (1,)), ((), ()))
        acc = jax.lax.dot_general(xs_a, ga_ref[...].astype(jnp.bfloat16), dn,
                                  preferred_element_type=jnp.float32)
        acc += jax.lax.dot_general(xs_b, gb_ref[...].astype(jnp.bfloat16), dn,
                                   preferred_element_type=jnp.float32)
        out_ref[...] = (acc + bias_ref[...]).astype(out_ref.dtype)
    return _mod_linear_kernel


def kernel(x, theta, gamma, bias):
    B, F = x.shape
    T, F2 = gamma.shape
    assert F == F2 and theta.shape == (F,) and bias.shape == (T,)
    dtype = x.dtype

    fh = _round_up(_cdiv(F, 2), 128)
    F_pad = 2 * fh
    T_pad = _round_up(T, 128)

    tm = min(1024, _round_up(B, 8))
    nb = _cdiv(B, tm)
    B_pad = nb * tm

    x_p = jnp.pad(x, ((0, B_pad - B), (0, F_pad - F)))
    gamma_p = jnp.pad(gamma, ((0, T_pad - T), (0, F_pad - F)))
    theta_p = jnp.pad(theta, (0, F_pad - F)).reshape(1, F_pad)
    bias_p = jnp.pad(bias, (0, T_pad - T)).reshape(1, T_pad)

    out = pl.pallas_call(
        _make_kernel(fh),
        out_shape=jax.ShapeDtypeStruct((B_pad, T_pad), dtype),
        grid=(nb,),
        in_specs=[
            pl.BlockSpec((tm, fh), lambda i: (i, 0)),
            pl.BlockSpec((tm, fh), lambda i: (i, 1)),
            pl.BlockSpec((1, F_pad), lambda i: (0, 0)),
            pl.BlockSpec((T_pad, fh), lambda i: (0, 0)),
            pl.BlockSpec((T_pad, fh), lambda i: (0, 1)),
            pl.BlockSpec((1, T_pad), lambda i: (0, 0)),
        ],
        out_specs=pl.BlockSpec((tm, T_pad), lambda i: (i, 0)),
        compiler_params=pltpu.CompilerParams(
            dimension_semantics=("parallel",),
            vmem_limit_bytes=48 * 1024 * 1024,
        ),
    )(x_p, x_p, theta_p, gamma_p, gamma_p, bias_p)

    return out[:B, :T]
